# Initial kernel scaffold; baseline (speedup 1.0000x reference)
import jax
import jax.numpy as jnp
from jax.experimental import pallas as pl


def kernel(x, edge_index, edge_weight, W_src_to_dst, W_dst_to_src, bias_src_to_dst, bias_dst_to_src):
    raise NotImplementedError("write your pallas kernel here")


# speedup vs baseline: 2.6878x; 2.6878x over previous
import functools

import jax
import jax.numpy as jnp
from jax import lax
from jax.experimental import pallas as pl
from jax.experimental.pallas import tpu as pltpu
from jax.experimental.pallas import tpu_sc as plsc

N_NODES = 10000
N_EDGES = 320000
D = 128

NC = 2
NS = 16
B = 128
G = 8
E_PAD = 327680
EB = E_PAD // B
TB = EB // NS
NG = TB // G
N_PAD = 10240
ROWS_PER_TEC = N_PAD // NS
RC = 128
RCHUNK = ROWS_PER_TEC // RC


def _sc_mesh():
    return plsc.VectorSubcoreMesh(
        core_axis_name="c", subcore_axis_name="s", num_cores=NC, num_subcores=NS
    )


@functools.partial(
    pl.kernel,
    out_type=(
        jax.ShapeDtypeStruct((N_PAD, D), jnp.float32),
        jax.ShapeDtypeStruct((N_PAD, D), jnp.float32),
    ),
    mesh=_sc_mesh(),
    scratch_types=[
        pltpu.VMEM_SHARED((N_PAD, D), jnp.float32),
        pltpu.VMEM((G, B), jnp.int32),
        pltpu.VMEM((G, B), jnp.int32),
        pltpu.VMEM((G, B), jnp.float32),
        pltpu.VMEM((B, D), jnp.float32),
        pltpu.SemaphoreType.DMA,
    ],
)
def _sc_agg(x_hbm, snd_hbm, rcv_hbm, w1_hbm, w2_hbm, out1_hbm, out2_hbm,
            acc, idx_v, rcv_v, w_v, rows_v, sem):
    cid = lax.axis_index("c")
    sid = lax.axis_index("s")

    def _zrow(i, _):
        for c in range(D // 16):
            rows_v[i, pl.ds(c * 16, 16)] = jnp.zeros((16,), jnp.float32)
        return 0

    lax.fori_loop(0, B, _zrow, 0)
    for k in range(RCHUNK):
        pltpu.sync_copy(rows_v, acc.at[pl.ds(sid * ROWS_PER_TEC + k * RC, RC)])
    plsc.subcore_barrier()

    def _group(g, _):
        base = sid * TB + g * G
        pltpu.sync_copy(snd_hbm.at[pl.ds(base, G)], idx_v)
        pltpu.sync_copy(rcv_hbm.at[pl.ds(base, G)], rcv_v)

        @pl.when(cid == 0)
        def _():
            pltpu.sync_copy(w1_hbm.at[pl.ds(base, G)], w_v)

        @pl.when(cid == 1)
        def _():
            pltpu.sync_copy(w2_hbm.at[pl.ds(base, G)], w_v)

        def _batch(j, _):
            pltpu.async_copy(x_hbm.at[idx_v.at[j]], rows_v, sem).wait()

            def _tile(rb, _):
                wvec = w_v[j, pl.ds(rb * 16, 16)]
                for l in range(16):
                    w = wvec[l]
                    r = rb * 16 + l
                    for c in range(D // 16):
                        sl = pl.ds(c * 16, 16)
                        rows_v[r, sl] = rows_v[r, sl] * w
                return 0

            lax.fori_loop(0, B // 16, _tile, 0)
            pltpu.sync_copy(rows_v, acc.at[rcv_v.at[j]], add=True)
            return 0

        lax.fori_loop(0, G, _batch, 0)
        return 0

    lax.fori_loop(0, NG, _group, 0)
    plsc.subcore_barrier()

    for k in range(RCHUNK):
        r0 = sid * ROWS_PER_TEC + k * RC
        pltpu.sync_copy(acc.at[pl.ds(r0, RC)], rows_v)

        @pl.when(cid == 0)
        def _():
            pltpu.sync_copy(rows_v, out1_hbm.at[pl.ds(r0, RC)])

        @pl.when(cid == 1)
        def _():
            pltpu.sync_copy(rows_v, out2_hbm.at[pl.ds(r0, RC)])


def _mm_body(a1, a2, w1, w2, b1, b2, o1, o2):
    o1[...] = jnp.dot(a1[...], w1[...], preferred_element_type=jnp.float32) + b1[...]
    o2[...] = jnp.dot(a2[...], w2[...], preferred_element_type=jnp.float32) + b2[...]


_MM_ROWS = 1000


def _dense(agg1, agg2, W1, W2, b1, b2):
    grid = (N_NODES // _MM_ROWS,)
    blk = pl.BlockSpec((_MM_ROWS, D), lambda i: (i, 0))
    wblk = pl.BlockSpec((D, D), lambda i: (0, 0))
    bblk = pl.BlockSpec((1, D), lambda i: (0, 0))
    return pl.pallas_call(
        _mm_body,
        grid=grid,
        in_specs=[blk, blk, wblk, wblk, bblk, bblk],
        out_specs=[blk, blk],
        out_shape=(
            jax.ShapeDtypeStruct((N_NODES, D), jnp.float32),
            jax.ShapeDtypeStruct((N_NODES, D), jnp.float32),
        ),
    )(agg1, agg2, W1, W2, b1, b2)


def kernel(x, edge_index, edge_weight, W_src_to_dst, W_dst_to_src,
           bias_src_to_dst, bias_dst_to_src):
    pad = E_PAD - N_EDGES
    snd = jnp.pad(edge_index[0].astype(jnp.int32), (0, pad)).reshape(EB, B)
    rcv = jnp.pad(edge_index[1].astype(jnp.int32), (0, pad)).reshape(EB, B)
    w1e = jnp.pad(edge_weight[0, :, 0].astype(jnp.float32), (0, pad)).reshape(EB, B)
    w2e = jnp.pad(edge_weight[1, :, 0].astype(jnp.float32), (0, pad)).reshape(EB, B)
    agg1, agg2 = _sc_agg(x, snd, rcv, w1e, w2e)
    agg1 = agg1[:N_NODES]
    agg2 = agg2[:N_NODES]
    return _dense(agg1, agg2, W_src_to_dst, W_dst_to_src,
                  bias_src_to_dst.reshape(1, D), bias_dst_to_src.reshape(1, D))
